# baseline (speedup 1.0000x reference)
import functools

import jax
import jax.numpy as jnp
from jax import lax
from jax.experimental import pallas as pl
from jax.experimental.pallas import tpu as pltpu
from jax.experimental.pallas import tpu_sc as plsc

N = 10000
E = 320000
D = 128
R = 3
C = 4

NC = 2
NS = 16
NHALF = N // NC
NLOCP = 5120
T = R * NLOCP
EPT = E // NS
G = 32
NBUF = 8
SUP = 512
NSUP = 40
EPT_PAD = (NSUP + 1) * SUP
ZROWS = 16

ACC_A = 2 * NLOCP
TPT_A = ACC_A // NS
TPT_B = NLOCP // NS


def _zero_buffers(zrow, zcnt):
    def zr(i, carry):
        zrow[i // 8, pl.ds((i % 8) * 16, 16)] = jnp.zeros((16,), jnp.float32)
        return carry
    lax.fori_loop(0, ZROWS * 8, zr, 0)

    def zc(i, carry):
        zcnt[pl.ds(i * 16, 16)] = jnp.zeros((16,), jnp.float32)
        return carry
    lax.fori_loop(0, TPT_A // 16, zc, 0)


def _phase(phase_b, s, nb, x, epack, acc_s, cnt_s,
           meta, rows, sidx, wv, gsem, ssem, csem, msem):
    iota = lax.iota(jnp.int32, 16)
    CPS = SUP // G

    def compute_chunk(j, mb, g, b):
        for k in range(G // 16):
            col = g * G + k * 16
            d16 = mb[1, col:col + 16]
            t16 = mb[2, col:col + 16]
            pos = j * SUP + col + iota
            valid = pos < EPT
            inhalf = (d16 >= nb) & (d16 < nb + NHALF)
            if phase_b:
                match = valid & inhalf & (t16 == 2)
                row = jnp.where(match, d16 - nb, NHALF + (d16 & 63))
            else:
                match = valid & inhalf & (t16 < 2)
                loc = jnp.where(match, d16 - nb, NHALF + (d16 & 63))
                row = jnp.where(match, t16, 0) * NLOCP + loc
            sidx[b][pl.ds(k * 16, 16)] = row
            wv[b][pl.ds(k * 16, 16)] = jnp.where(
                match, jnp.float32(1.0), jnp.float32(0.0))

    for b in (NBUF - 2, NBUF - 1):
        for k in range(G // 16):
            sidx[b][pl.ds(k * 16, 16)] = jnp.full((16,), NHALF, jnp.int32)
            wv[b][pl.ds(k * 16, 16)] = jnp.zeros((16,), jnp.float32)
        pltpu.async_copy(rows[b], acc_s.at[sidx[b]], ssem[b], add=True)
        pltpu.async_copy(wv[b], cnt_s.at[sidx[b]], csem[b], add=True)

    pltpu.sync_copy(epack.at[s, :, pl.ds(0, SUP)], meta[0])
    for i in range(NBUF - 2):
        pltpu.async_copy(x.at[meta[0].at[0, pl.ds(i * G, G)]], rows[i],
                         gsem[i])

    def super_pair(j2, carry):
        for jj in range(2):
            j = j2 * 2 + jj
            mb = meta[jj]
            mbn = meta[1 - jj]
            pltpu.async_copy(epack.at[s, :, pl.ds((j + 1) * SUP, SUP)],
                             mbn, msem)
            for g in range(CPS):
                b = g % NBUF
                tb = (g + NBUF - 2) % NBUF
                pf = g + NBUF - 2
                pltpu.make_async_copy(
                    x.at[mb.at[0, pl.ds(g * G, G)]], rows[b],
                    gsem[b]).wait()
                compute_chunk(j, mb, g, b)
                pltpu.make_async_copy(rows[tb], acc_s.at[sidx[tb]],
                                      ssem[tb]).wait()
                pltpu.make_async_copy(wv[tb], cnt_s.at[sidx[tb]],
                                      csem[tb]).wait()
                if pf < CPS:
                    pltpu.async_copy(x.at[mb.at[0, pl.ds(pf * G, G)]],
                                     rows[tb], gsem[tb])
                else:
                    if pf == CPS:
                        pltpu.make_async_copy(
                            epack.at[s, :, pl.ds((j + 1) * SUP, SUP)],
                            mbn, msem).wait()
                    pltpu.async_copy(
                        x.at[mbn.at[0, pl.ds((pf - CPS) * G, G)]],
                        rows[tb], gsem[tb])
                pltpu.async_copy(rows[b], acc_s.at[sidx[b]], ssem[b],
                                 add=True)
                pltpu.async_copy(wv[b], cnt_s.at[sidx[b]], csem[b],
                                 add=True)
        return carry

    lax.fori_loop(0, NSUP // 2, super_pair, 0)

    for i in range(NBUF - 2):
        pltpu.make_async_copy(x.at[meta[0].at[0, pl.ds(i * G, G)]],
                              rows[i], gsem[i]).wait()
    for b in (NBUF - 2, NBUF - 1):
        pltpu.make_async_copy(rows[b], acc_s.at[sidx[b]], ssem[b]).wait()
        pltpu.make_async_copy(wv[b], cnt_s.at[sidx[b]], csem[b]).wait()


def _sc_tile(epack, x, acc_out, cnt_out, acc_s, cnt_s, *scr):
    meta = scr[0:2]
    rows = scr[2:2 + NBUF]
    sidx = scr[2 + NBUF:2 + 2 * NBUF]
    wv = scr[2 + 2 * NBUF:2 + 3 * NBUF]
    zrow = scr[2 + 3 * NBUF]
    zcnt = scr[3 + 3 * NBUF]
    gsem = scr[4 + 3 * NBUF:4 + 4 * NBUF]
    ssem = scr[4 + 4 * NBUF:4 + 5 * NBUF]
    csem = scr[4 + 5 * NBUF:4 + 6 * NBUF]
    msem = scr[4 + 6 * NBUF]

    c = lax.axis_index("c")
    s = lax.axis_index("s")
    nb = c * NHALF

    _zero_buffers(zrow, zcnt)

    def za(t, carry):
        pltpu.sync_copy(zrow, acc_s.at[pl.ds(s * TPT_A + t * ZROWS, ZROWS)])
        return carry
    lax.fori_loop(0, TPT_A // ZROWS, za, 0)
    pltpu.sync_copy(zcnt, cnt_s.at[pl.ds(s * TPT_A, TPT_A)])
    plsc.subcore_barrier()

    _phase(False, s, nb, x, epack, acc_s, cnt_s,
           meta, rows, sidx, wv, gsem, ssem, csem, msem)
    plsc.subcore_barrier()

    def cpa(t, carry):
        pltpu.sync_copy(acc_s.at[pl.ds(s * TPT_A + t * ZROWS, ZROWS)], zrow)
        pltpu.sync_copy(zrow,
                        acc_out.at[c, pl.ds(s * TPT_A + t * ZROWS, ZROWS)])
        return carry
    lax.fori_loop(0, TPT_A // ZROWS, cpa, 0)
    pltpu.sync_copy(cnt_s.at[pl.ds(s * TPT_A, TPT_A)], zcnt)
    pltpu.sync_copy(zcnt, cnt_out.at[pl.ds(c * T + s * TPT_A, TPT_A)])
    plsc.subcore_barrier()

    _zero_buffers(zrow, zcnt)

    def zb(t, carry):
        pltpu.sync_copy(zrow, acc_s.at[pl.ds(s * TPT_B + t * ZROWS, ZROWS)])
        return carry
    lax.fori_loop(0, TPT_B // ZROWS, zb, 0)
    pltpu.sync_copy(zcnt.at[pl.ds(0, TPT_B)],
                    cnt_s.at[pl.ds(s * TPT_B, TPT_B)])
    plsc.subcore_barrier()

    _phase(True, s, nb, x, epack, acc_s, cnt_s,
           meta, rows, sidx, wv, gsem, ssem, csem, msem)
    plsc.subcore_barrier()

    def cpb(t, carry):
        pltpu.sync_copy(acc_s.at[pl.ds(s * TPT_B + t * ZROWS, ZROWS)], zrow)
        pltpu.sync_copy(
            zrow, acc_out.at[c, pl.ds(ACC_A + s * TPT_B + t * ZROWS, ZROWS)])
        return carry
    lax.fori_loop(0, TPT_B // ZROWS, cpb, 0)
    pltpu.sync_copy(cnt_s.at[pl.ds(s * TPT_B, TPT_B)],
                    zcnt.at[pl.ds(0, TPT_B)])
    pltpu.sync_copy(zcnt.at[pl.ds(0, TPT_B)],
                    cnt_out.at[pl.ds(c * T + ACC_A + s * TPT_B, TPT_B)])


def _sc_body(epack, x, acc_out, cnt_out, acc_s, cnt_s):
    scratch = (
        [pltpu.VMEM((3, SUP), jnp.int32)] * 2
        + [pltpu.VMEM((G, D), jnp.float32)] * NBUF
        + [pltpu.VMEM((G,), jnp.int32)] * NBUF
        + [pltpu.VMEM((G,), jnp.float32)] * NBUF
        + [pltpu.VMEM((ZROWS, D), jnp.float32)]
        + [pltpu.VMEM((TPT_A,), jnp.float32)]
        + [pltpu.SemaphoreType.DMA] * NBUF
        + [pltpu.SemaphoreType.DMA] * NBUF
        + [pltpu.SemaphoreType.DMA] * NBUF
        + [pltpu.SemaphoreType.DMA]
    )
    pl.run_scoped(
        functools.partial(_sc_tile, epack, x, acc_out, cnt_out,
                          acc_s, cnt_s),
        *scratch,
    )


_MESH = plsc.VectorSubcoreMesh(core_axis_name="c", subcore_axis_name="s")

_sc_scatter = functools.partial(
    pl.kernel,
    mesh=_MESH,
    compiler_params=pltpu.CompilerParams(use_tc_tiling_on_sc=False),
    out_type=[
        jax.ShapeDtypeStruct((NC, T, D), jnp.float32),
        jax.ShapeDtypeStruct((NC * T,), jnp.float32),
    ],
    scratch_types=[
        pltpu.VMEM_SHARED((ACC_A, D), jnp.float32) @ _MESH,
        pltpu.VMEM_SHARED((ACC_A,), jnp.float32) @ _MESH,
    ],
)(_sc_body)


def _tc_body(x_ref, acc_ref, cnt_ref, W1_ref, root1_ref, b1_ref,
             Wout_ref, bout_ref, o_ref):
    xb = x_ref[...]
    h = jnp.dot(xb, root1_ref[...], preferred_element_type=jnp.float32)
    h = h + b1_ref[0]
    cnt = cnt_ref[0].reshape(T)
    for r in range(R):
        A = acc_ref[0, r * NLOCP:r * NLOCP + NHALF, :]
        cr = jnp.maximum(cnt[r * NLOCP:r * NLOCP + NHALF], 1.0)
        h = h + jnp.dot(A / cr[:, None], W1_ref[r],
                        preferred_element_type=jnp.float32)
    h = jnp.maximum(h, 0.0)
    o_ref[...] = jnp.dot(h, Wout_ref[...],
                         preferred_element_type=jnp.float32) + bout_ref[0]


def kernel(x_content, edge_index, edge_type, W0, root0, b0,
           W1, root1, b1, Wout, bout):
    src = edge_index[0]
    dst = edge_index[1]

    def padtile(a):
        return jnp.pad(a.reshape(NS, EPT), ((0, 0), (0, EPT_PAD - EPT)))

    epack = jnp.stack(
        [padtile(src), padtile(dst), padtile(edge_type)], axis=1)

    acc, cnt = _sc_scatter(epack, x_content)
    cnt3 = cnt.reshape(NC, T // 128, 128)
    out = pl.pallas_call(
        _tc_body,
        grid=(NC,),
        in_specs=[
            pl.BlockSpec((NHALF, D), lambda c: (c, 0)),
            pl.BlockSpec((1, T, D), lambda c: (c, 0, 0)),
            pl.BlockSpec((1, T // 128, 128), lambda c: (c, 0, 0)),
            pl.BlockSpec((R, D, D), lambda c: (0, 0, 0)),
            pl.BlockSpec((D, D), lambda c: (0, 0)),
            pl.BlockSpec((1, D), lambda c: (0, 0)),
            pl.BlockSpec((D, C), lambda c: (0, 0)),
            pl.BlockSpec((1, C), lambda c: (0, 0)),
        ],
        out_specs=pl.BlockSpec((NHALF, C), lambda c: (c, 0)),
        out_shape=jax.ShapeDtypeStruct((N, C), jnp.float32),
    )(x_content, acc, cnt3, W1, root1, b1.reshape(1, D),
      Wout, bout.reshape(1, C))
    return out


# speedup vs baseline: 4.6327x; 4.6327x over previous
import functools

import jax
import jax.numpy as jnp
from jax import lax
from jax.experimental import pallas as pl
from jax.experimental.pallas import tpu as pltpu
from jax.experimental.pallas import tpu_sc as plsc

N = 10000
E = 320000
D = 128
R = 3
C = 4

NC = 2
NS = 16
NHALF = N // NC
NLOCP = 5120
T = R * NLOCP
EPT = E // NS
G = 128
QCAP = G + 32
SUP = 512
NSUP = 40
EPT_PAD = (NSUP + 1) * SUP
ZROWS = 16

ACC_A = 2 * NLOCP
TPT_A = ACC_A // NS
TPT_B = NLOCP // NS


def _zero_buffers(zrow, zcnt):
    def zr(i, carry):
        zrow[i // 8, pl.ds((i % 8) * 16, 16)] = jnp.zeros((16,), jnp.float32)
        return carry
    lax.fori_loop(0, ZROWS * 8, zr, 0)

    def zc(i, carry):
        zcnt[pl.ds(i * 16, 16)] = jnp.zeros((16,), jnp.float32)
        return carry
    lax.fori_loop(0, TPT_A // 16, zc, 0)


def _phase(phase_b, s, nb, x, epack, acc_s, cnt_s, meta, rows, gidx,
           sidq, qsrc, qsid, stg_s, stg_d, ones, gsem, ssem, csem, msem):
    iota = lax.iota(jnp.int32, 16)
    SPS = SUP // 16

    def fire_parity(p, fcnt):
        for k in range(G // 16):
            gidx[p][pl.ds(k * 16, 16)] = qsrc[pl.ds(k * 16, 16)]
            sidq[p][pl.ds(k * 16, 16)] = qsid[pl.ds(k * 16, 16)]
        @pl.when(fcnt >= 2)
        def _():
            pltpu.make_async_copy(rows[p], acc_s.at[sidq[p]],
                                  ssem[p]).wait()
            pltpu.make_async_copy(ones, cnt_s.at[sidq[p]], csem[p]).wait()
        pltpu.async_copy(x.at[gidx[p]], rows[p], gsem[p]).wait()
        pltpu.async_copy(rows[p], acc_s.at[sidq[p]], ssem[p], add=True)
        pltpu.async_copy(ones, cnt_s.at[sidq[p]], csem[p], add=True)

    def fire(fcnt):
        @pl.when(lax.rem(fcnt, 2) == 0)
        def _():
            fire_parity(0, fcnt)

        @pl.when(lax.rem(fcnt, 2) == 1)
        def _():
            fire_parity(1, fcnt)
        qsrc[pl.ds(0, 16)] = qsrc[pl.ds(G, 16)]
        qsid[pl.ds(0, 16)] = qsid[pl.ds(G, 16)]

    def scan_step(i, mb, j, qn, fcnt):
        col = i * 16
        s16 = mb[0, pl.ds(col, 16)]
        d16 = mb[1, pl.ds(col, 16)]
        t16 = mb[2, pl.ds(col, 16)]
        pos = j * SUP + col + iota
        valid = pos < EPT
        inhalf = (d16 >= nb) & (d16 < nb + NHALF)
        if phase_b:
            match = valid & inhalf & (t16 == 2)
            sid = d16 - nb
        else:
            match = valid & inhalf & (t16 < 2)
            sid = t16 * NLOCP + (d16 - nb)
        plsc.store_compressed(stg_s.at[pl.ds(0, 16)], s16, mask=match)
        plsc.store_compressed(stg_d.at[pl.ds(0, 16)], sid, mask=match)
        qsrc[pl.ds(qn, 16)] = stg_s[pl.ds(0, 16)]
        qsid[pl.ds(qn, 16)] = stg_d[pl.ds(0, 16)]
        qn = qn + jnp.max(plsc.all_reduce_population_count(match))
        fire_pred = qn >= G
        pl.when(fire_pred)(lambda: fire(fcnt))
        qn = jnp.where(fire_pred, qn - G, qn)
        fcnt = fcnt + fire_pred.astype(jnp.int32)
        return qn, fcnt

    pltpu.sync_copy(epack.at[s, :, pl.ds(0, SUP)], meta[0])

    def super_pair(j2, carry):
        qn, fcnt = carry
        for jj in range(2):
            j = j2 * 2 + jj
            mb = meta[jj]
            mbn = meta[1 - jj]
            pltpu.async_copy(epack.at[s, :, pl.ds((j + 1) * SUP, SUP)],
                             mbn, msem)

            def step(i, c):
                return scan_step(i, mb, j, *c)
            qn, fcnt = lax.fori_loop(0, SPS, step, (qn, fcnt))
            pltpu.make_async_copy(epack.at[s, :, pl.ds((j + 1) * SUP, SUP)],
                                  mbn, msem).wait()
        return qn, fcnt

    qn, fcnt = lax.fori_loop(0, NSUP // 2, super_pair,
                             (jnp.int32(0), jnp.int32(0)))

    for k in range(G // 16):
        posk = k * 16 + iota
        keep = posk < qn
        gq = jnp.where(keep, qsrc[pl.ds(k * 16, 16)], 0)
        sq = jnp.where(keep, qsid[pl.ds(k * 16, 16)],
                       NHALF + (posk & 63))
        qsrc[pl.ds(k * 16, 16)] = gq
        qsid[pl.ds(k * 16, 16)] = sq
    fire(fcnt)
    fcnt = fcnt + 1

    pltpu.make_async_copy(rows[0], acc_s.at[sidq[0]], ssem[0]).wait()
    pltpu.make_async_copy(ones, cnt_s.at[sidq[0]], csem[0]).wait()

    @pl.when(fcnt >= 2)
    def _():
        pltpu.make_async_copy(rows[1], acc_s.at[sidq[1]], ssem[1]).wait()
        pltpu.make_async_copy(ones, cnt_s.at[sidq[1]], csem[1]).wait()


def _sc_tile(epack, x, acc_out, cnt_out, acc_s, cnt_s, *scr):
    meta = scr[0:2]
    rows = scr[2:4]
    gidx = scr[4:6]
    sidq = scr[6:8]
    qsrc = scr[8]
    qsid = scr[9]
    stg_s = scr[10]
    stg_d = scr[11]
    ones = scr[12]
    zrow = scr[13]
    zcnt = scr[14]
    gsem = scr[15:17]
    ssem = scr[17:19]
    csem = scr[19:21]
    msem = scr[21]

    c = lax.axis_index("c")
    s = lax.axis_index("s")
    nb = c * NHALF

    _zero_buffers(zrow, zcnt)

    def oinit(i, carry):
        ones[pl.ds(i * 16, 16)] = jnp.ones((16,), jnp.float32)
        return carry
    lax.fori_loop(0, G // 16, oinit, 0)

    def za(t, carry):
        pltpu.sync_copy(zrow, acc_s.at[pl.ds(s * TPT_A + t * ZROWS, ZROWS)])
        return carry
    lax.fori_loop(0, TPT_A // ZROWS, za, 0)
    pltpu.sync_copy(zcnt, cnt_s.at[pl.ds(s * TPT_A, TPT_A)])
    plsc.subcore_barrier()

    _phase(False, s, nb, x, epack, acc_s, cnt_s, meta, rows, gidx,
           sidq, qsrc, qsid, stg_s, stg_d, ones, gsem, ssem, csem, msem)
    plsc.subcore_barrier()

    def cpa(t, carry):
        pltpu.sync_copy(acc_s.at[pl.ds(s * TPT_A + t * ZROWS, ZROWS)], zrow)
        pltpu.sync_copy(zrow,
                        acc_out.at[c, pl.ds(s * TPT_A + t * ZROWS, ZROWS)])
        return carry
    lax.fori_loop(0, TPT_A // ZROWS, cpa, 0)
    pltpu.sync_copy(cnt_s.at[pl.ds(s * TPT_A, TPT_A)], zcnt)
    pltpu.sync_copy(zcnt, cnt_out.at[pl.ds(c * T + s * TPT_A, TPT_A)])
    plsc.subcore_barrier()

    _zero_buffers(zrow, zcnt)

    def zb(t, carry):
        pltpu.sync_copy(zrow, acc_s.at[pl.ds(s * TPT_B + t * ZROWS, ZROWS)])
        return carry
    lax.fori_loop(0, TPT_B // ZROWS, zb, 0)
    pltpu.sync_copy(zcnt.at[pl.ds(0, TPT_B)],
                    cnt_s.at[pl.ds(s * TPT_B, TPT_B)])
    plsc.subcore_barrier()

    _phase(True, s, nb, x, epack, acc_s, cnt_s, meta, rows, gidx,
           sidq, qsrc, qsid, stg_s, stg_d, ones, gsem, ssem, csem, msem)
    plsc.subcore_barrier()

    def cpb(t, carry):
        pltpu.sync_copy(acc_s.at[pl.ds(s * TPT_B + t * ZROWS, ZROWS)], zrow)
        pltpu.sync_copy(
            zrow, acc_out.at[c, pl.ds(ACC_A + s * TPT_B + t * ZROWS, ZROWS)])
        return carry
    lax.fori_loop(0, TPT_B // ZROWS, cpb, 0)
    pltpu.sync_copy(cnt_s.at[pl.ds(s * TPT_B, TPT_B)],
                    zcnt.at[pl.ds(0, TPT_B)])
    pltpu.sync_copy(zcnt.at[pl.ds(0, TPT_B)],
                    cnt_out.at[pl.ds(c * T + ACC_A + s * TPT_B, TPT_B)])


def _sc_body(epack, x, acc_out, cnt_out, acc_s, cnt_s):
    scratch = (
        [pltpu.VMEM((3, SUP), jnp.int32)] * 2
        + [pltpu.VMEM((G, D), jnp.float32)] * 2
        + [pltpu.VMEM((G,), jnp.int32)] * 2
        + [pltpu.VMEM((G,), jnp.int32)] * 2
        + [pltpu.VMEM((QCAP,), jnp.int32)]
        + [pltpu.VMEM((QCAP,), jnp.int32)]
        + [pltpu.VMEM((16,), jnp.int32)]
        + [pltpu.VMEM((16,), jnp.int32)]
        + [pltpu.VMEM((G,), jnp.float32)]
        + [pltpu.VMEM((ZROWS, D), jnp.float32)]
        + [pltpu.VMEM((TPT_A,), jnp.float32)]
        + [pltpu.SemaphoreType.DMA] * 2
        + [pltpu.SemaphoreType.DMA] * 2
        + [pltpu.SemaphoreType.DMA] * 2
        + [pltpu.SemaphoreType.DMA]
    )
    pl.run_scoped(
        functools.partial(_sc_tile, epack, x, acc_out, cnt_out,
                          acc_s, cnt_s),
        *scratch,
    )


_MESH = plsc.VectorSubcoreMesh(core_axis_name="c", subcore_axis_name="s")

_sc_scatter = functools.partial(
    pl.kernel,
    mesh=_MESH,
    compiler_params=pltpu.CompilerParams(needs_layout_passes=False),
    out_type=[
        jax.ShapeDtypeStruct((NC, T, D), jnp.float32),
        jax.ShapeDtypeStruct((NC * T,), jnp.float32),
    ],
    scratch_types=[
        pltpu.VMEM_SHARED((ACC_A, D), jnp.float32) @ _MESH,
        pltpu.VMEM_SHARED((ACC_A,), jnp.float32) @ _MESH,
    ],
)(_sc_body)


def _tc_body(x_ref, acc_ref, cnt_ref, W1_ref, root1_ref, b1_ref,
             Wout_ref, bout_ref, o_ref):
    xb = x_ref[...]
    h = jnp.dot(xb, root1_ref[...], preferred_element_type=jnp.float32)
    h = h + b1_ref[0]
    cnt = cnt_ref[0].reshape(T)
    for r in range(R):
        A = acc_ref[0, r * NLOCP:r * NLOCP + NHALF, :]
        cr = jnp.maximum(cnt[r * NLOCP:r * NLOCP + NHALF], 1.0)
        h = h + jnp.dot(A / cr[:, None], W1_ref[r],
                        preferred_element_type=jnp.float32)
    h = jnp.maximum(h, 0.0)
    o_ref[...] = jnp.dot(h, Wout_ref[...],
                         preferred_element_type=jnp.float32) + bout_ref[0]


def kernel(x_content, edge_index, edge_type, W0, root0, b0,
           W1, root1, b1, Wout, bout):
    src = edge_index[0]
    dst = edge_index[1]

    def padtile(a):
        return jnp.pad(a.reshape(NS, EPT), ((0, 0), (0, EPT_PAD - EPT)))

    epack = jnp.stack(
        [padtile(src), padtile(dst), padtile(edge_type)], axis=1)

    acc, cnt = _sc_scatter(epack, x_content)
    cnt3 = cnt.reshape(NC, T // 128, 128)
    out = pl.pallas_call(
        _tc_body,
        grid=(NC,),
        in_specs=[
            pl.BlockSpec((NHALF, D), lambda c: (c, 0)),
            pl.BlockSpec((1, T, D), lambda c: (c, 0, 0)),
            pl.BlockSpec((1, T // 128, 128), lambda c: (c, 0, 0)),
            pl.BlockSpec((R, D, D), lambda c: (0, 0, 0)),
            pl.BlockSpec((D, D), lambda c: (0, 0)),
            pl.BlockSpec((1, D), lambda c: (0, 0)),
            pl.BlockSpec((D, C), lambda c: (0, 0)),
            pl.BlockSpec((1, C), lambda c: (0, 0)),
        ],
        out_specs=pl.BlockSpec((NHALF, C), lambda c: (c, 0)),
        out_shape=jax.ShapeDtypeStruct((N, C), jnp.float32),
    )(x_content, acc, cnt3, W1, root1, b1.reshape(1, D),
      Wout, bout.reshape(1, C))
    return out


# speedup vs baseline: 5.6627x; 1.2223x over previous
import functools

import jax
import jax.numpy as jnp
from jax import lax
from jax.experimental import pallas as pl
from jax.experimental.pallas import tpu as pltpu
from jax.experimental.pallas import tpu_sc as plsc

N = 10000
E = 320000
D = 128
R = 3
C = 4

NC = 2
NS = 16
NHALF = N // NC
NLOCP = 5120
T = R * NLOCP
EPT = E // NS
G = 128
QCAP = G + 32
SUP = 512
NSUP = 40
EPT_PAD = (NSUP + 1) * SUP
ZROWS = 64

ACC_A = 2 * NLOCP
TPT_A = ACC_A // NS
TPT_B = NLOCP // NS


def _zero_buffers(zrow, zcnt):
    def zr(i, carry):
        zrow[i // 8, pl.ds((i % 8) * 16, 16)] = jnp.zeros((16,), jnp.float32)
        return carry
    lax.fori_loop(0, ZROWS * 8, zr, 0)

    def zc(i, carry):
        zcnt[pl.ds(i * 16, 16)] = jnp.zeros((16,), jnp.float32)
        return carry
    lax.fori_loop(0, TPT_A // 16, zc, 0)


def _phase(phase_b, s, nb, x, epack, acc_s, cnt_s, meta, rows, gidx,
           sidq, qsrc, qsid, stg_s, stg_d, ones, gsem, ssem, csem, msem):
    iota = lax.iota(jnp.int32, 16)
    SPS = SUP // 16

    def fire_parity(p, fcnt):
        @pl.when(fcnt >= 2)
        def _():
            pltpu.make_async_copy(rows[p], acc_s.at[sidq[p]],
                                  ssem[p]).wait()
            pltpu.make_async_copy(ones, cnt_s.at[sidq[p]], csem[p]).wait()
        for k in range(G // 16):
            gidx[p][pl.ds(k * 16, 16)] = qsrc[pl.ds(k * 16, 16)]
            sidq[p][pl.ds(k * 16, 16)] = qsid[pl.ds(k * 16, 16)]
        pltpu.async_copy(x.at[gidx[p]], rows[p], gsem[p])
        @pl.when(fcnt >= 1)
        def _():
            pltpu.make_async_copy(x.at[gidx[1 - p]], rows[1 - p],
                                  gsem[1 - p]).wait()
            pltpu.async_copy(rows[1 - p], acc_s.at[sidq[1 - p]],
                             ssem[1 - p], add=True)
            pltpu.async_copy(ones, cnt_s.at[sidq[1 - p]], csem[1 - p],
                             add=True)

    def fire(fcnt):
        @pl.when(lax.rem(fcnt, 2) == 0)
        def _():
            fire_parity(0, fcnt)

        @pl.when(lax.rem(fcnt, 2) == 1)
        def _():
            fire_parity(1, fcnt)
        qsrc[pl.ds(0, 16)] = qsrc[pl.ds(G, 16)]
        qsid[pl.ds(0, 16)] = qsid[pl.ds(G, 16)]

    def scan_step(i, mb, j, qn, fcnt):
        col = i * 16
        s16 = mb[0, pl.ds(col, 16)]
        d16 = mb[1, pl.ds(col, 16)]
        t16 = mb[2, pl.ds(col, 16)]
        pos = j * SUP + col + iota
        valid = pos < EPT
        inhalf = (d16 >= nb) & (d16 < nb + NHALF)
        if phase_b:
            match = valid & inhalf & (t16 == 2)
            sid = d16 - nb
        else:
            match = valid & inhalf & (t16 < 2)
            sid = t16 * NLOCP + (d16 - nb)
        plsc.store_compressed(stg_s.at[pl.ds(0, 16)], s16, mask=match)
        plsc.store_compressed(stg_d.at[pl.ds(0, 16)], sid, mask=match)
        qsrc[pl.ds(qn, 16)] = stg_s[pl.ds(0, 16)]
        qsid[pl.ds(qn, 16)] = stg_d[pl.ds(0, 16)]
        qn = qn + jnp.max(plsc.all_reduce_population_count(match))
        fire_pred = qn >= G
        pl.when(fire_pred)(lambda: fire(fcnt))
        qn = jnp.where(fire_pred, qn - G, qn)
        fcnt = fcnt + fire_pred.astype(jnp.int32)
        return qn, fcnt

    pltpu.sync_copy(epack.at[s, :, pl.ds(0, SUP)], meta[0])

    def super_pair(j2, carry):
        qn, fcnt = carry
        for jj in range(2):
            j = j2 * 2 + jj
            mb = meta[jj]
            mbn = meta[1 - jj]
            pltpu.async_copy(epack.at[s, :, pl.ds((j + 1) * SUP, SUP)],
                             mbn, msem)

            def step(i, c):
                return scan_step(i, mb, j, *c)
            qn, fcnt = lax.fori_loop(0, SPS, step, (qn, fcnt))
            pltpu.make_async_copy(epack.at[s, :, pl.ds((j + 1) * SUP, SUP)],
                                  mbn, msem).wait()
        return qn, fcnt

    qn, fcnt = lax.fori_loop(0, NSUP // 2, super_pair,
                             (jnp.int32(0), jnp.int32(0)))

    for k in range(G // 16):
        posk = k * 16 + iota
        keep = posk < qn
        gq = jnp.where(keep, qsrc[pl.ds(k * 16, 16)], 0)
        sq = jnp.where(keep, qsid[pl.ds(k * 16, 16)],
                       NHALF + (posk & 63))
        qsrc[pl.ds(k * 16, 16)] = gq
        qsid[pl.ds(k * 16, 16)] = sq
    fire(fcnt)
    last = lax.rem(fcnt, 2)
    fcnt = fcnt + 1

    @pl.when(last == 0)
    def _():
        pltpu.make_async_copy(x.at[gidx[0]], rows[0], gsem[0]).wait()
        pltpu.async_copy(rows[0], acc_s.at[sidq[0]], ssem[0], add=True)
        pltpu.async_copy(ones, cnt_s.at[sidq[0]], csem[0], add=True)

    @pl.when(last == 1)
    def _():
        pltpu.make_async_copy(x.at[gidx[1]], rows[1], gsem[1]).wait()
        pltpu.async_copy(rows[1], acc_s.at[sidq[1]], ssem[1], add=True)
        pltpu.async_copy(ones, cnt_s.at[sidq[1]], csem[1], add=True)

    pltpu.make_async_copy(rows[0], acc_s.at[sidq[0]], ssem[0]).wait()
    pltpu.make_async_copy(ones, cnt_s.at[sidq[0]], csem[0]).wait()

    @pl.when(fcnt >= 2)
    def _():
        pltpu.make_async_copy(rows[1], acc_s.at[sidq[1]], ssem[1]).wait()
        pltpu.make_async_copy(ones, cnt_s.at[sidq[1]], csem[1]).wait()


def _sc_tile(epack, x, acc_out, cnt_out, acc_s, cnt_s, *scr):
    meta = scr[0:2]
    rows = scr[2:4]
    gidx = scr[4:6]
    sidq = scr[6:8]
    qsrc = scr[8]
    qsid = scr[9]
    stg_s = scr[10]
    stg_d = scr[11]
    ones = scr[12]
    zrow = scr[13]
    zcnt = scr[14]
    gsem = scr[15:17]
    ssem = scr[17:19]
    csem = scr[19:21]
    msem = scr[21]

    c = lax.axis_index("c")
    s = lax.axis_index("s")
    nb = c * NHALF

    _zero_buffers(zrow, zcnt)

    def oinit(i, carry):
        ones[pl.ds(i * 16, 16)] = jnp.ones((16,), jnp.float32)
        return carry
    lax.fori_loop(0, G // 16, oinit, 0)

    def za(t, carry):
        pltpu.sync_copy(zrow, acc_s.at[pl.ds(s * TPT_A + t * ZROWS, ZROWS)])
        return carry
    lax.fori_loop(0, TPT_A // ZROWS, za, 0)
    pltpu.sync_copy(zcnt, cnt_s.at[pl.ds(s * TPT_A, TPT_A)])
    plsc.subcore_barrier()

    _phase(False, s, nb, x, epack, acc_s, cnt_s, meta, rows, gidx,
           sidq, qsrc, qsid, stg_s, stg_d, ones, gsem, ssem, csem, msem)
    plsc.subcore_barrier()

    def cpa(t, carry):
        pltpu.sync_copy(acc_s.at[pl.ds(s * TPT_A + t * ZROWS, ZROWS)], zrow)
        pltpu.sync_copy(zrow,
                        acc_out.at[c, pl.ds(s * TPT_A + t * ZROWS, ZROWS)])
        return carry
    lax.fori_loop(0, TPT_A // ZROWS, cpa, 0)
    pltpu.sync_copy(cnt_s.at[pl.ds(s * TPT_A, TPT_A)], zcnt)
    pltpu.sync_copy(zcnt, cnt_out.at[pl.ds(c * T + s * TPT_A, TPT_A)])
    plsc.subcore_barrier()

    _zero_buffers(zrow, zcnt)

    def zb(t, carry):
        pltpu.sync_copy(zrow, acc_s.at[pl.ds(s * TPT_B + t * ZROWS, ZROWS)])
        return carry
    lax.fori_loop(0, TPT_B // ZROWS, zb, 0)
    pltpu.sync_copy(zcnt.at[pl.ds(0, TPT_B)],
                    cnt_s.at[pl.ds(s * TPT_B, TPT_B)])
    plsc.subcore_barrier()

    _phase(True, s, nb, x, epack, acc_s, cnt_s, meta, rows, gidx,
           sidq, qsrc, qsid, stg_s, stg_d, ones, gsem, ssem, csem, msem)
    plsc.subcore_barrier()

    def cpb(t, carry):
        pltpu.sync_copy(acc_s.at[pl.ds(s * TPT_B + t * ZROWS, ZROWS)], zrow)
        pltpu.sync_copy(
            zrow, acc_out.at[c, pl.ds(ACC_A + s * TPT_B + t * ZROWS, ZROWS)])
        return carry
    lax.fori_loop(0, TPT_B // ZROWS, cpb, 0)
    pltpu.sync_copy(cnt_s.at[pl.ds(s * TPT_B, TPT_B)],
                    zcnt.at[pl.ds(0, TPT_B)])
    pltpu.sync_copy(zcnt.at[pl.ds(0, TPT_B)],
                    cnt_out.at[pl.ds(c * T + ACC_A + s * TPT_B, TPT_B)])


def _sc_body(epack, x, acc_out, cnt_out, acc_s, cnt_s):
    scratch = (
        [pltpu.VMEM((3, SUP), jnp.int32)] * 2
        + [pltpu.VMEM((G, D), jnp.float32)] * 2
        + [pltpu.VMEM((G,), jnp.int32)] * 2
        + [pltpu.VMEM((G,), jnp.int32)] * 2
        + [pltpu.VMEM((QCAP,), jnp.int32)]
        + [pltpu.VMEM((QCAP,), jnp.int32)]
        + [pltpu.VMEM((16,), jnp.int32)]
        + [pltpu.VMEM((16,), jnp.int32)]
        + [pltpu.VMEM((G,), jnp.float32)]
        + [pltpu.VMEM((ZROWS, D), jnp.float32)]
        + [pltpu.VMEM((TPT_A,), jnp.float32)]
        + [pltpu.SemaphoreType.DMA] * 2
        + [pltpu.SemaphoreType.DMA] * 2
        + [pltpu.SemaphoreType.DMA] * 2
        + [pltpu.SemaphoreType.DMA]
    )
    pl.run_scoped(
        functools.partial(_sc_tile, epack, x, acc_out, cnt_out,
                          acc_s, cnt_s),
        *scratch,
    )


_MESH = plsc.VectorSubcoreMesh(core_axis_name="c", subcore_axis_name="s")

_sc_scatter = functools.partial(
    pl.kernel,
    mesh=_MESH,
    compiler_params=pltpu.CompilerParams(needs_layout_passes=False),
    out_type=[
        jax.ShapeDtypeStruct((NC, T, D), jnp.float32),
        jax.ShapeDtypeStruct((NC * T,), jnp.float32),
    ],
    scratch_types=[
        pltpu.VMEM_SHARED((ACC_A, D), jnp.float32) @ _MESH,
        pltpu.VMEM_SHARED((ACC_A,), jnp.float32) @ _MESH,
    ],
)(_sc_body)


def _tc_body(x_ref, acc_ref, cnt_ref, W1_ref, root1_ref, b1_ref,
             Wout_ref, bout_ref, o_ref):
    xb = x_ref[...]
    h = jnp.dot(xb, root1_ref[...], preferred_element_type=jnp.float32)
    h = h + b1_ref[0]
    cnt = cnt_ref[0].reshape(T)
    for r in range(R):
        A = acc_ref[0, r * NLOCP:r * NLOCP + NHALF, :]
        cr = jnp.maximum(cnt[r * NLOCP:r * NLOCP + NHALF], 1.0)
        h = h + jnp.dot(A / cr[:, None], W1_ref[r],
                        preferred_element_type=jnp.float32)
    h = jnp.maximum(h, 0.0)
    o_ref[...] = jnp.dot(h, Wout_ref[...],
                         preferred_element_type=jnp.float32) + bout_ref[0]


def kernel(x_content, edge_index, edge_type, W0, root0, b0,
           W1, root1, b1, Wout, bout):
    src = edge_index[0]
    dst = edge_index[1]

    def padtile(a):
        return jnp.pad(a.reshape(NS, EPT), ((0, 0), (0, EPT_PAD - EPT)))

    epack = jnp.stack(
        [padtile(src), padtile(dst), padtile(edge_type)], axis=1)

    acc, cnt = _sc_scatter(epack, x_content)
    cnt3 = cnt.reshape(NC, T // 128, 128)
    out = pl.pallas_call(
        _tc_body,
        grid=(NC,),
        in_specs=[
            pl.BlockSpec((NHALF, D), lambda c: (c, 0)),
            pl.BlockSpec((1, T, D), lambda c: (c, 0, 0)),
            pl.BlockSpec((1, T // 128, 128), lambda c: (c, 0, 0)),
            pl.BlockSpec((R, D, D), lambda c: (0, 0, 0)),
            pl.BlockSpec((D, D), lambda c: (0, 0)),
            pl.BlockSpec((1, D), lambda c: (0, 0)),
            pl.BlockSpec((D, C), lambda c: (0, 0)),
            pl.BlockSpec((1, C), lambda c: (0, 0)),
        ],
        out_specs=pl.BlockSpec((NHALF, C), lambda c: (c, 0)),
        out_shape=jax.ShapeDtypeStruct((N, C), jnp.float32),
    )(x_content, acc, cnt3, W1, root1, b1.reshape(1, D),
      Wout, bout.reshape(1, C))
    return out


# speedup vs baseline: 7.0470x; 1.2445x over previous
import functools

import jax
import jax.numpy as jnp
from jax import lax
from jax.experimental import pallas as pl
from jax.experimental.pallas import tpu as pltpu
from jax.experimental.pallas import tpu_sc as plsc

N = 10000
E = 320000
D = 128
R = 3
C = 4

NC = 2
NS = 16
NHALF = N // NC
NLOCP = 5120
T = R * NLOCP
EPT = E // NS
G = 96
NRING = 3
QCAP = G + 32
SUP = 512
NSUP = 40
EPT_PAD = (NSUP + 1) * SUP
ZROWS = 32

ACC_A = 2 * NLOCP
TPT_A = ACC_A // NS
TPT_B = NLOCP // NS


def _zero_buffers(zrow, zcnt):
    def zr(i, carry):
        zrow[i // 8, pl.ds((i % 8) * 16, 16)] = jnp.zeros((16,), jnp.float32)
        return carry
    lax.fori_loop(0, ZROWS * 8, zr, 0)

    def zc(i, carry):
        zcnt[pl.ds(i * 16, 16)] = jnp.zeros((16,), jnp.float32)
        return carry
    lax.fori_loop(0, TPT_A // 16, zc, 0)


def _phase(phase_b, s, nb, x, epack, acc_s, cnt_s, meta, rows, gidx,
           sidq, qsrc, qsid, stg_s, stg_d, ones, gsem, ssem, csem, msem):
    iota = lax.iota(jnp.int32, 16)
    SPS = SUP // 16

    def fire_parity(p, fcnt):
        @pl.when(fcnt >= NRING)
        def _():
            pltpu.make_async_copy(rows[p], acc_s.at[sidq[p]],
                                  ssem[p]).wait()
            pltpu.make_async_copy(ones, cnt_s.at[sidq[p]], csem[p]).wait()
        for k in range(G // 16):
            gidx[p][pl.ds(k * 16, 16)] = qsrc[pl.ds(k * 16, 16)]
            sidq[p][pl.ds(k * 16, 16)] = qsid[pl.ds(k * 16, 16)]
        pltpu.async_copy(x.at[gidx[p]], rows[p], gsem[p])
        p2 = (p + 1) % NRING
        @pl.when(fcnt >= 2)
        def _():
            pltpu.make_async_copy(x.at[gidx[p2]], rows[p2],
                                  gsem[p2]).wait()
            pltpu.async_copy(rows[p2], acc_s.at[sidq[p2]],
                             ssem[p2], add=True)
            pltpu.async_copy(ones, cnt_s.at[sidq[p2]], csem[p2],
                             add=True)

    def fire(fcnt):
        for p in range(NRING):
            @pl.when(lax.rem(fcnt, NRING) == p)
            def _(p=p):
                fire_parity(p, fcnt)
        qsrc[pl.ds(0, 16)] = qsrc[pl.ds(G, 16)]
        qsid[pl.ds(0, 16)] = qsid[pl.ds(G, 16)]

    def scan_step(i, mb, j, qn, fcnt):
        col = i * 16
        s16 = mb[0, pl.ds(col, 16)]
        d16 = mb[1, pl.ds(col, 16)]
        t16 = mb[2, pl.ds(col, 16)]
        pos = j * SUP + col + iota
        valid = pos < EPT
        inhalf = (d16 >= nb) & (d16 < nb + NHALF)
        if phase_b:
            match = valid & inhalf & (t16 == 2)
            sid = d16 - nb
        else:
            match = valid & inhalf & (t16 < 2)
            sid = t16 * NLOCP + (d16 - nb)
        plsc.store_compressed(stg_s.at[pl.ds(0, 16)], s16, mask=match)
        plsc.store_compressed(stg_d.at[pl.ds(0, 16)], sid, mask=match)
        qsrc[pl.ds(qn, 16)] = stg_s[pl.ds(0, 16)]
        qsid[pl.ds(qn, 16)] = stg_d[pl.ds(0, 16)]
        qn = qn + jnp.max(plsc.all_reduce_population_count(match))
        fire_pred = qn >= G
        pl.when(fire_pred)(lambda: fire(fcnt))
        qn = jnp.where(fire_pred, qn - G, qn)
        fcnt = fcnt + fire_pred.astype(jnp.int32)
        return qn, fcnt

    pltpu.sync_copy(epack.at[s, :, pl.ds(0, SUP)], meta[0])

    def super_pair(j2, carry):
        qn, fcnt = carry
        for jj in range(2):
            j = j2 * 2 + jj
            mb = meta[jj]
            mbn = meta[1 - jj]
            pltpu.async_copy(epack.at[s, :, pl.ds((j + 1) * SUP, SUP)],
                             mbn, msem)

            def step(i, c):
                return scan_step(i, mb, j, *c)
            qn, fcnt = lax.fori_loop(0, SPS, step, (qn, fcnt))
            pltpu.make_async_copy(epack.at[s, :, pl.ds((j + 1) * SUP, SUP)],
                                  mbn, msem).wait()
        return qn, fcnt

    qn, fcnt = lax.fori_loop(0, NSUP // 2, super_pair,
                             (jnp.int32(0), jnp.int32(0)))

    for k in range(G // 16):
        posk = k * 16 + iota
        keep = posk < qn
        gq = jnp.where(keep, qsrc[pl.ds(k * 16, 16)], 0)
        sq = jnp.where(keep, qsid[pl.ds(k * 16, 16)],
                       NHALF + (posk & 63))
        qsrc[pl.ds(k * 16, 16)] = gq
        qsid[pl.ds(k * 16, 16)] = sq
    fire(fcnt)
    fcnt = fcnt + 1

    for p in range(NRING):
        @pl.when((fcnt >= 1) & (lax.rem(fcnt - 1, NRING) == p))
        def _(p=p):
            pltpu.make_async_copy(x.at[gidx[p]], rows[p], gsem[p]).wait()
            pltpu.async_copy(rows[p], acc_s.at[sidq[p]], ssem[p],
                             add=True)
            pltpu.async_copy(ones, cnt_s.at[sidq[p]], csem[p], add=True)

        @pl.when((fcnt >= 2) & (lax.rem(fcnt - 2, NRING) == p))
        def _(p=p):
            pltpu.make_async_copy(x.at[gidx[p]], rows[p], gsem[p]).wait()
            pltpu.async_copy(rows[p], acc_s.at[sidq[p]], ssem[p],
                             add=True)
            pltpu.async_copy(ones, cnt_s.at[sidq[p]], csem[p], add=True)
    for p in range(NRING):
        @pl.when(fcnt >= p + 1)
        def _(p=p):
            pltpu.make_async_copy(rows[p], acc_s.at[sidq[p]],
                                  ssem[p]).wait()
            pltpu.make_async_copy(ones, cnt_s.at[sidq[p]], csem[p]).wait()


def _sc_tile(epack, x, acc_out, cnt_out, acc_s, cnt_s, *scr):
    meta = scr[0:2]
    rows = scr[2:2 + NRING]
    gidx = scr[2 + NRING:2 + 2 * NRING]
    sidq = scr[2 + 2 * NRING:2 + 3 * NRING]
    qsrc = scr[2 + 3 * NRING]
    qsid = scr[3 + 3 * NRING]
    stg_s = scr[4 + 3 * NRING]
    stg_d = scr[5 + 3 * NRING]
    ones = scr[6 + 3 * NRING]
    zrow = scr[7 + 3 * NRING]
    zcnt = scr[8 + 3 * NRING]
    gsem = scr[9 + 3 * NRING:9 + 4 * NRING]
    ssem = scr[9 + 4 * NRING:9 + 5 * NRING]
    csem = scr[9 + 5 * NRING:9 + 6 * NRING]
    msem = scr[9 + 6 * NRING]

    c = lax.axis_index("c")
    s = lax.axis_index("s")
    nb = c * NHALF

    _zero_buffers(zrow, zcnt)

    def oinit(i, carry):
        ones[pl.ds(i * 16, 16)] = jnp.ones((16,), jnp.float32)
        return carry
    lax.fori_loop(0, G // 16, oinit, 0)

    def za(t, carry):
        pltpu.sync_copy(zrow, acc_s.at[pl.ds(s * TPT_A + t * ZROWS, ZROWS)])
        return carry
    lax.fori_loop(0, TPT_A // ZROWS, za, 0)
    pltpu.sync_copy(zcnt, cnt_s.at[pl.ds(s * TPT_A, TPT_A)])
    plsc.subcore_barrier()

    _phase(False, s, nb, x, epack, acc_s, cnt_s, meta, rows, gidx,
           sidq, qsrc, qsid, stg_s, stg_d, ones, gsem, ssem, csem, msem)
    plsc.subcore_barrier()

    def cpa(t, carry):
        pltpu.sync_copy(acc_s.at[pl.ds(s * TPT_A + t * ZROWS, ZROWS)], zrow)
        pltpu.sync_copy(zrow,
                        acc_out.at[c, pl.ds(s * TPT_A + t * ZROWS, ZROWS)])
        return carry
    lax.fori_loop(0, TPT_A // ZROWS, cpa, 0)
    pltpu.sync_copy(cnt_s.at[pl.ds(s * TPT_A, TPT_A)], zcnt)
    pltpu.sync_copy(zcnt, cnt_out.at[pl.ds(c * T + s * TPT_A, TPT_A)])
    plsc.subcore_barrier()

    _zero_buffers(zrow, zcnt)

    def zb(t, carry):
        pltpu.sync_copy(zrow, acc_s.at[pl.ds(s * TPT_B + t * ZROWS, ZROWS)])
        return carry
    lax.fori_loop(0, TPT_B // ZROWS, zb, 0)
    pltpu.sync_copy(zcnt.at[pl.ds(0, TPT_B)],
                    cnt_s.at[pl.ds(s * TPT_B, TPT_B)])
    plsc.subcore_barrier()

    _phase(True, s, nb, x, epack, acc_s, cnt_s, meta, rows, gidx,
           sidq, qsrc, qsid, stg_s, stg_d, ones, gsem, ssem, csem, msem)
    plsc.subcore_barrier()

    def cpb(t, carry):
        pltpu.sync_copy(acc_s.at[pl.ds(s * TPT_B + t * ZROWS, ZROWS)], zrow)
        pltpu.sync_copy(
            zrow, acc_out.at[c, pl.ds(ACC_A + s * TPT_B + t * ZROWS, ZROWS)])
        return carry
    lax.fori_loop(0, TPT_B // ZROWS, cpb, 0)
    pltpu.sync_copy(cnt_s.at[pl.ds(s * TPT_B, TPT_B)],
                    zcnt.at[pl.ds(0, TPT_B)])
    pltpu.sync_copy(zcnt.at[pl.ds(0, TPT_B)],
                    cnt_out.at[pl.ds(c * T + ACC_A + s * TPT_B, TPT_B)])


def _sc_body(epack, x, acc_out, cnt_out, acc_s, cnt_s):
    scratch = (
        [pltpu.VMEM((3, SUP), jnp.int32)] * 2
        + [pltpu.VMEM((G, D), jnp.float32)] * NRING
        + [pltpu.VMEM((G,), jnp.int32)] * NRING
        + [pltpu.VMEM((G,), jnp.int32)] * NRING
        + [pltpu.VMEM((QCAP,), jnp.int32)]
        + [pltpu.VMEM((QCAP,), jnp.int32)]
        + [pltpu.VMEM((16,), jnp.int32)]
        + [pltpu.VMEM((16,), jnp.int32)]
        + [pltpu.VMEM((G,), jnp.float32)]
        + [pltpu.VMEM((ZROWS, D), jnp.float32)]
        + [pltpu.VMEM((TPT_A,), jnp.float32)]
        + [pltpu.SemaphoreType.DMA] * NRING
        + [pltpu.SemaphoreType.DMA] * NRING
        + [pltpu.SemaphoreType.DMA] * NRING
        + [pltpu.SemaphoreType.DMA]
    )
    pl.run_scoped(
        functools.partial(_sc_tile, epack, x, acc_out, cnt_out,
                          acc_s, cnt_s),
        *scratch,
    )


_MESH = plsc.VectorSubcoreMesh(core_axis_name="c", subcore_axis_name="s")

_sc_scatter = functools.partial(
    pl.kernel,
    mesh=_MESH,
    compiler_params=pltpu.CompilerParams(needs_layout_passes=False),
    out_type=[
        jax.ShapeDtypeStruct((NC, T, D), jnp.float32),
        jax.ShapeDtypeStruct((NC * T,), jnp.float32),
    ],
    scratch_types=[
        pltpu.VMEM_SHARED((ACC_A, D), jnp.float32) @ _MESH,
        pltpu.VMEM_SHARED((ACC_A,), jnp.float32) @ _MESH,
    ],
)(_sc_body)


def _tc_body(x_ref, acc_ref, cnt_ref, W1_ref, root1_ref, b1_ref,
             Wout_ref, bout_ref, o_ref):
    xb = x_ref[...]
    h = jnp.dot(xb, root1_ref[...], preferred_element_type=jnp.float32)
    h = h + b1_ref[0]
    cnt = cnt_ref[0].reshape(T)
    for r in range(R):
        A = acc_ref[0, r * NLOCP:r * NLOCP + NHALF, :]
        cr = jnp.maximum(cnt[r * NLOCP:r * NLOCP + NHALF], 1.0)
        h = h + jnp.dot(A / cr[:, None], W1_ref[r],
                        preferred_element_type=jnp.float32)
    h = jnp.maximum(h, 0.0)
    o_ref[...] = jnp.dot(h, Wout_ref[...],
                         preferred_element_type=jnp.float32) + bout_ref[0]


def kernel(x_content, edge_index, edge_type, W0, root0, b0,
           W1, root1, b1, Wout, bout):
    src = edge_index[0]
    dst = edge_index[1]

    def padtile(a):
        return jnp.pad(a.reshape(NS, EPT), ((0, 0), (0, EPT_PAD - EPT)))

    epack = jnp.stack(
        [padtile(src), padtile(dst), padtile(edge_type)], axis=1)

    acc, cnt = _sc_scatter(epack, x_content)
    cnt3 = cnt.reshape(NC, T // 128, 128)
    out = pl.pallas_call(
        _tc_body,
        grid=(NC,),
        in_specs=[
            pl.BlockSpec((NHALF, D), lambda c: (c, 0)),
            pl.BlockSpec((1, T, D), lambda c: (c, 0, 0)),
            pl.BlockSpec((1, T // 128, 128), lambda c: (c, 0, 0)),
            pl.BlockSpec((R, D, D), lambda c: (0, 0, 0)),
            pl.BlockSpec((D, D), lambda c: (0, 0)),
            pl.BlockSpec((1, D), lambda c: (0, 0)),
            pl.BlockSpec((D, C), lambda c: (0, 0)),
            pl.BlockSpec((1, C), lambda c: (0, 0)),
        ],
        out_specs=pl.BlockSpec((NHALF, C), lambda c: (c, 0)),
        out_shape=jax.ShapeDtypeStruct((N, C), jnp.float32),
    )(x_content, acc, cnt3, W1, root1, b1.reshape(1, D),
      Wout, bout.reshape(1, C))
    return out


# speedup vs baseline: 7.5115x; 1.0659x over previous
import functools

import jax
import jax.numpy as jnp
from jax import lax
from jax.experimental import pallas as pl
from jax.experimental.pallas import tpu as pltpu
from jax.experimental.pallas import tpu_sc as plsc

N = 10000
E = 320000
D = 128
R = 3
C = 4

NC = 2
NS = 16
NHALF = N // NC
NLOCP = 5120
T = R * NLOCP
EPT = E // NS
G = 64
NRING = 4
QCAP = G + 32
SUP = 512
NSUP = 40
EPT_PAD = (NSUP + 1) * SUP
ZROWS = 32

ACC_A = 2 * NLOCP
TPT_A = ACC_A // NS
TPT_B = NLOCP // NS


def _zero_buffers(zrow, zcnt):
    def zr(i, carry):
        zrow[i // 8, pl.ds((i % 8) * 16, 16)] = jnp.zeros((16,), jnp.float32)
        return carry
    lax.fori_loop(0, ZROWS * 8, zr, 0)

    def zc(i, carry):
        zcnt[pl.ds(i * 16, 16)] = jnp.zeros((16,), jnp.float32)
        return carry
    lax.fori_loop(0, TPT_A // 16, zc, 0)


def _phase(phase_b, s, nb, x, epack, acc_s, cnt_s, meta, rows, gidx,
           sidq, qsrc, qsid, stg_s, stg_d, ones, gsem, ssem, csem, msem):
    iota = lax.iota(jnp.int32, 16)
    SPS = SUP // 16

    def fire_parity(p, fcnt):
        @pl.when(fcnt >= NRING)
        def _():
            pltpu.make_async_copy(rows[p], acc_s.at[sidq[p]],
                                  ssem[p]).wait()
            pltpu.make_async_copy(ones, cnt_s.at[sidq[p]], csem[p]).wait()
        for k in range(G // 16):
            gidx[p][pl.ds(k * 16, 16)] = qsrc[pl.ds(k * 16, 16)]
            sidq[p][pl.ds(k * 16, 16)] = qsid[pl.ds(k * 16, 16)]
        pltpu.async_copy(x.at[gidx[p]], rows[p], gsem[p])
        p2 = (p + NRING - 2) % NRING
        @pl.when(fcnt >= 2)
        def _():
            pltpu.make_async_copy(x.at[gidx[p2]], rows[p2],
                                  gsem[p2]).wait()
            pltpu.async_copy(rows[p2], acc_s.at[sidq[p2]],
                             ssem[p2], add=True)
            pltpu.async_copy(ones, cnt_s.at[sidq[p2]], csem[p2],
                             add=True)

    def fire(fcnt):
        for p in range(NRING):
            @pl.when(lax.rem(fcnt, NRING) == p)
            def _(p=p):
                fire_parity(p, fcnt)
        qsrc[pl.ds(0, 16)] = qsrc[pl.ds(G, 16)]
        qsid[pl.ds(0, 16)] = qsid[pl.ds(G, 16)]

    def scan_step(i, mb, j, qn, fcnt):
        col = i * 16
        s16 = mb[0, pl.ds(col, 16)]
        d16 = mb[1, pl.ds(col, 16)]
        t16 = mb[2, pl.ds(col, 16)]
        pos = j * SUP + col + iota
        valid = pos < EPT
        inhalf = (d16 >= nb) & (d16 < nb + NHALF)
        if phase_b:
            match = valid & inhalf & (t16 == 2)
            sid = d16 - nb
        else:
            match = valid & inhalf & (t16 < 2)
            sid = t16 * NLOCP + (d16 - nb)
        plsc.store_compressed(stg_s.at[pl.ds(0, 16)], s16, mask=match)
        plsc.store_compressed(stg_d.at[pl.ds(0, 16)], sid, mask=match)
        qsrc[pl.ds(qn, 16)] = stg_s[pl.ds(0, 16)]
        qsid[pl.ds(qn, 16)] = stg_d[pl.ds(0, 16)]
        qn = qn + jnp.max(plsc.all_reduce_population_count(match))
        fire_pred = qn >= G
        pl.when(fire_pred)(lambda: fire(fcnt))
        qn = jnp.where(fire_pred, qn - G, qn)
        fcnt = fcnt + fire_pred.astype(jnp.int32)
        return qn, fcnt

    pltpu.sync_copy(epack.at[s, :, pl.ds(0, SUP)], meta[0])

    def super_pair(j2, carry):
        qn, fcnt = carry
        for jj in range(2):
            j = j2 * 2 + jj
            mb = meta[jj]
            mbn = meta[1 - jj]
            pltpu.async_copy(epack.at[s, :, pl.ds((j + 1) * SUP, SUP)],
                             mbn, msem)

            def step(i, c):
                return scan_step(i, mb, j, *c)
            qn, fcnt = lax.fori_loop(0, SPS, step, (qn, fcnt))
            pltpu.make_async_copy(epack.at[s, :, pl.ds((j + 1) * SUP, SUP)],
                                  mbn, msem).wait()
        return qn, fcnt

    qn, fcnt = lax.fori_loop(0, NSUP // 2, super_pair,
                             (jnp.int32(0), jnp.int32(0)))

    for k in range(G // 16):
        posk = k * 16 + iota
        keep = posk < qn
        gq = jnp.where(keep, qsrc[pl.ds(k * 16, 16)], 0)
        sq = jnp.where(keep, qsid[pl.ds(k * 16, 16)],
                       NHALF + (posk & 63))
        qsrc[pl.ds(k * 16, 16)] = gq
        qsid[pl.ds(k * 16, 16)] = sq
    fire(fcnt)
    fcnt = fcnt + 1

    for p in range(NRING):
        @pl.when((fcnt >= 1) & (lax.rem(fcnt - 1, NRING) == p))
        def _(p=p):
            pltpu.make_async_copy(x.at[gidx[p]], rows[p], gsem[p]).wait()
            pltpu.async_copy(rows[p], acc_s.at[sidq[p]], ssem[p],
                             add=True)
            pltpu.async_copy(ones, cnt_s.at[sidq[p]], csem[p], add=True)

        @pl.when((fcnt >= 2) & (lax.rem(fcnt - 2, NRING) == p))
        def _(p=p):
            pltpu.make_async_copy(x.at[gidx[p]], rows[p], gsem[p]).wait()
            pltpu.async_copy(rows[p], acc_s.at[sidq[p]], ssem[p],
                             add=True)
            pltpu.async_copy(ones, cnt_s.at[sidq[p]], csem[p], add=True)
    for p in range(NRING):
        @pl.when(fcnt >= p + 1)
        def _(p=p):
            pltpu.make_async_copy(rows[p], acc_s.at[sidq[p]],
                                  ssem[p]).wait()
            pltpu.make_async_copy(ones, cnt_s.at[sidq[p]], csem[p]).wait()


def _sc_tile(epack, x, acc_out, cnt_out, acc_s, cnt_s, *scr):
    meta = scr[0:2]
    rows = scr[2:2 + NRING]
    gidx = scr[2 + NRING:2 + 2 * NRING]
    sidq = scr[2 + 2 * NRING:2 + 3 * NRING]
    qsrc = scr[2 + 3 * NRING]
    qsid = scr[3 + 3 * NRING]
    stg_s = scr[4 + 3 * NRING]
    stg_d = scr[5 + 3 * NRING]
    ones = scr[6 + 3 * NRING]
    zrow = scr[7 + 3 * NRING]
    zcnt = scr[8 + 3 * NRING]
    gsem = scr[9 + 3 * NRING:9 + 4 * NRING]
    ssem = scr[9 + 4 * NRING:9 + 5 * NRING]
    csem = scr[9 + 5 * NRING:9 + 6 * NRING]
    msem = scr[9 + 6 * NRING]

    c = lax.axis_index("c")
    s = lax.axis_index("s")
    nb = c * NHALF

    _zero_buffers(zrow, zcnt)

    def oinit(i, carry):
        ones[pl.ds(i * 16, 16)] = jnp.ones((16,), jnp.float32)
        return carry
    lax.fori_loop(0, G // 16, oinit, 0)

    def za(t, carry):
        pltpu.sync_copy(zrow, acc_s.at[pl.ds(s * TPT_A + t * ZROWS, ZROWS)])
        return carry
    lax.fori_loop(0, TPT_A // ZROWS, za, 0)
    pltpu.sync_copy(zcnt, cnt_s.at[pl.ds(s * TPT_A, TPT_A)])
    plsc.subcore_barrier()

    _phase(False, s, nb, x, epack, acc_s, cnt_s, meta, rows, gidx,
           sidq, qsrc, qsid, stg_s, stg_d, ones, gsem, ssem, csem, msem)
    plsc.subcore_barrier()

    def cpa(t, carry):
        pltpu.sync_copy(acc_s.at[pl.ds(s * TPT_A + t * ZROWS, ZROWS)], zrow)
        pltpu.sync_copy(zrow,
                        acc_out.at[c, pl.ds(s * TPT_A + t * ZROWS, ZROWS)])
        return carry
    lax.fori_loop(0, TPT_A // ZROWS, cpa, 0)
    pltpu.sync_copy(cnt_s.at[pl.ds(s * TPT_A, TPT_A)], zcnt)
    pltpu.sync_copy(zcnt, cnt_out.at[pl.ds(c * T + s * TPT_A, TPT_A)])
    plsc.subcore_barrier()

    _zero_buffers(zrow, zcnt)

    def zb(t, carry):
        pltpu.sync_copy(zrow, acc_s.at[pl.ds(s * TPT_B + t * ZROWS, ZROWS)])
        return carry
    lax.fori_loop(0, TPT_B // ZROWS, zb, 0)
    pltpu.sync_copy(zcnt.at[pl.ds(0, TPT_B)],
                    cnt_s.at[pl.ds(s * TPT_B, TPT_B)])
    plsc.subcore_barrier()

    _phase(True, s, nb, x, epack, acc_s, cnt_s, meta, rows, gidx,
           sidq, qsrc, qsid, stg_s, stg_d, ones, gsem, ssem, csem, msem)
    plsc.subcore_barrier()

    def cpb(t, carry):
        pltpu.sync_copy(acc_s.at[pl.ds(s * TPT_B + t * ZROWS, ZROWS)], zrow)
        pltpu.sync_copy(
            zrow, acc_out.at[c, pl.ds(ACC_A + s * TPT_B + t * ZROWS, ZROWS)])
        return carry
    lax.fori_loop(0, TPT_B // ZROWS, cpb, 0)
    pltpu.sync_copy(cnt_s.at[pl.ds(s * TPT_B, TPT_B)],
                    zcnt.at[pl.ds(0, TPT_B)])
    pltpu.sync_copy(zcnt.at[pl.ds(0, TPT_B)],
                    cnt_out.at[pl.ds(c * T + ACC_A + s * TPT_B, TPT_B)])


def _sc_body(epack, x, acc_out, cnt_out, acc_s, cnt_s):
    scratch = (
        [pltpu.VMEM((3, SUP), jnp.int32)] * 2
        + [pltpu.VMEM((G, D), jnp.float32)] * NRING
        + [pltpu.VMEM((G,), jnp.int32)] * NRING
        + [pltpu.VMEM((G,), jnp.int32)] * NRING
        + [pltpu.VMEM((QCAP,), jnp.int32)]
        + [pltpu.VMEM((QCAP,), jnp.int32)]
        + [pltpu.VMEM((16,), jnp.int32)]
        + [pltpu.VMEM((16,), jnp.int32)]
        + [pltpu.VMEM((G,), jnp.float32)]
        + [pltpu.VMEM((ZROWS, D), jnp.float32)]
        + [pltpu.VMEM((TPT_A,), jnp.float32)]
        + [pltpu.SemaphoreType.DMA] * NRING
        + [pltpu.SemaphoreType.DMA] * NRING
        + [pltpu.SemaphoreType.DMA] * NRING
        + [pltpu.SemaphoreType.DMA]
    )
    pl.run_scoped(
        functools.partial(_sc_tile, epack, x, acc_out, cnt_out,
                          acc_s, cnt_s),
        *scratch,
    )


_MESH = plsc.VectorSubcoreMesh(core_axis_name="c", subcore_axis_name="s")

_sc_scatter = functools.partial(
    pl.kernel,
    mesh=_MESH,
    compiler_params=pltpu.CompilerParams(needs_layout_passes=False),
    out_type=[
        jax.ShapeDtypeStruct((NC, T, D), jnp.float32),
        jax.ShapeDtypeStruct((NC * T,), jnp.float32),
    ],
    scratch_types=[
        pltpu.VMEM_SHARED((ACC_A, D), jnp.float32) @ _MESH,
        pltpu.VMEM_SHARED((ACC_A,), jnp.float32) @ _MESH,
    ],
)(_sc_body)


def _tc_body(x_ref, acc_ref, cnt_ref, W1_ref, root1_ref, b1_ref,
             Wout_ref, bout_ref, o_ref):
    xb = x_ref[...]
    h = jnp.dot(xb, root1_ref[...], preferred_element_type=jnp.float32)
    h = h + b1_ref[0]
    cnt = cnt_ref[0].reshape(T)
    for r in range(R):
        A = acc_ref[0, r * NLOCP:r * NLOCP + NHALF, :]
        cr = jnp.maximum(cnt[r * NLOCP:r * NLOCP + NHALF], 1.0)
        h = h + jnp.dot(A / cr[:, None], W1_ref[r],
                        preferred_element_type=jnp.float32)
    h = jnp.maximum(h, 0.0)
    o_ref[...] = jnp.dot(h, Wout_ref[...],
                         preferred_element_type=jnp.float32) + bout_ref[0]


def kernel(x_content, edge_index, edge_type, W0, root0, b0,
           W1, root1, b1, Wout, bout):
    src = edge_index[0]
    dst = edge_index[1]

    def padtile(a):
        return jnp.pad(a.reshape(NS, EPT), ((0, 0), (0, EPT_PAD - EPT)))

    epack = jnp.stack(
        [padtile(src), padtile(dst), padtile(edge_type)], axis=1)

    acc, cnt = _sc_scatter(epack, x_content)
    cnt3 = cnt.reshape(NC, T // 128, 128)
    out = pl.pallas_call(
        _tc_body,
        grid=(NC,),
        in_specs=[
            pl.BlockSpec((NHALF, D), lambda c: (c, 0)),
            pl.BlockSpec((1, T, D), lambda c: (c, 0, 0)),
            pl.BlockSpec((1, T // 128, 128), lambda c: (c, 0, 0)),
            pl.BlockSpec((R, D, D), lambda c: (0, 0, 0)),
            pl.BlockSpec((D, D), lambda c: (0, 0)),
            pl.BlockSpec((1, D), lambda c: (0, 0)),
            pl.BlockSpec((D, C), lambda c: (0, 0)),
            pl.BlockSpec((1, C), lambda c: (0, 0)),
        ],
        out_specs=pl.BlockSpec((NHALF, C), lambda c: (c, 0)),
        out_shape=jax.ShapeDtypeStruct((N, C), jnp.float32),
    )(x_content, acc, cnt3, W1, root1, b1.reshape(1, D),
      Wout, bout.reshape(1, C))
    return out
